# Initial kernel scaffold; baseline (speedup 1.0000x reference)
import jax
import jax.numpy as jnp
from jax.experimental import pallas as pl


def kernel(input, char_emb, conv_w, conv_b, w_proj, b_proj, w_gate, b_gate):
    raise NotImplementedError("write your pallas kernel here")


# speedup vs baseline: 1.8331x; 1.8331x over previous
import functools

import jax
import jax.numpy as jnp
from jax import lax
from jax.experimental import pallas as pl
from jax.experimental.pallas import tpu as pltpu
from jax.experimental.pallas import tpu_sc as plsc

S, B, M = 256, 128, 21
V_CHAR, E_CHAR, E_WORD = 96, 30, 128
KW, PAD = 5, 1
N = S * B
T_OUT = M - KW + 1 + 2 * PAD
EC_PAD = 32

NC, NS = 2, 16
NW = NC * NS
TOT = N * M
CHUNK = 128
NCHUNK = TOT // (NW * CHUNK)


def _sc_gather(table, idx):
    mesh = plsc.VectorSubcoreMesh(core_axis_name="c", subcore_axis_name="s")

    @functools.partial(
        pl.kernel,
        mesh=mesh,
        compiler_params=pltpu.CompilerParams(use_tc_tiling_on_sc=False),
        out_type=jax.ShapeDtypeStruct((TOT, EC_PAD), jnp.float32),
        scratch_types=[
            pltpu.VMEM((NCHUNK, CHUNK), jnp.int32),
            pltpu.VMEM((CHUNK, EC_PAD), jnp.float32),
            pltpu.SemaphoreType.DMA,
        ],
    )
    def run(table_hbm, idx_hbm, out_hbm, idx_v, rows_v, sem):
        wid = lax.axis_index("s") * NC + lax.axis_index("c")
        pltpu.sync_copy(idx_hbm.at[wid], idx_v)
        base = wid * (NCHUNK * CHUNK)

        def body(j, carry):
            pltpu.async_copy(table_hbm.at[idx_v.at[j]], rows_v, sem).wait()
            pltpu.sync_copy(rows_v, out_hbm.at[pl.ds(base + j * CHUNK, CHUNK)])
            return carry

        lax.fori_loop(0, NCHUNK, body, 0)

    return run(table, idx)


def _tc_body(x_ref, wu_ref, cb_ref, whw_ref, bhw_ref, o_ref, *, nb):
    x = x_ref[...]
    x3 = x.reshape(nb, M, EC_PAD)
    z = jnp.zeros((nb, 1, EC_PAD), dtype=x.dtype)
    pieces = [
        jnp.concatenate([z, x3[:, 0:18]], axis=1),
        x3[:, 0:19],
        x3[:, 1:20],
        x3[:, 2:21],
        jnp.concatenate([x3[:, 3:21], z], axis=1),
    ]
    xu = jnp.concatenate(pieces, axis=2).reshape(nb * T_OUT, KW * EC_PAD)
    y = jnp.dot(xu, wu_ref[...], preferred_element_type=jnp.float32)
    y = jnp.maximum(y + cb_ref[...], 0.0)
    xc = jnp.max(y.reshape(nb, T_OUT, E_WORD), axis=1)
    hw = jnp.dot(xc, whw_ref[...], preferred_element_type=jnp.float32) + bhw_ref[...]
    proj = jnp.maximum(hw[:, :E_WORD], 0.0)
    gate = 1.0 / (1.0 + jnp.exp(-hw[:, E_WORD:]))
    o_ref[...] = gate * proj + (1.0 - gate) * xc


def kernel(input, char_emb, conv_w, conv_b, w_proj, b_proj, w_gate, b_gate):
    nb = 256
    ce = jnp.pad(char_emb, ((0, 0), (0, EC_PAD - E_CHAR)))
    wu = jnp.pad(conv_w, ((0, 0), (0, EC_PAD - E_CHAR), (0, 0))).reshape(KW * EC_PAD, E_WORD)
    whw = jnp.concatenate([w_proj.T, w_gate.T], axis=1)
    bhw = jnp.concatenate([b_proj, b_gate])[None, :]
    cb = conv_b[None, :]

    idx = input.reshape(NW, NCHUNK, CHUNK)
    x_emb = _sc_gather(ce, idx)

    out = pl.pallas_call(
        functools.partial(_tc_body, nb=nb),
        grid=(N // nb,),
        in_specs=[
            pl.BlockSpec((nb * M, EC_PAD), lambda i: (i, 0)),
            pl.BlockSpec((KW * EC_PAD, E_WORD), lambda i: (0, 0)),
            pl.BlockSpec((1, E_WORD), lambda i: (0, 0)),
            pl.BlockSpec((E_WORD, 2 * E_WORD), lambda i: (0, 0)),
            pl.BlockSpec((1, 2 * E_WORD), lambda i: (0, 0)),
        ],
        out_specs=pl.BlockSpec((nb, E_WORD), lambda i: (i, 0)),
        out_shape=jax.ShapeDtypeStruct((N, E_WORD), jnp.float32),
    )(x_emb, wu, cb, whw, bhw)
    return out.reshape(S, B, E_WORD)


# speedup vs baseline: 2.6094x; 1.4235x over previous
import functools

import jax
import jax.numpy as jnp
from jax import lax
from jax.experimental import pallas as pl
from jax.experimental.pallas import tpu as pltpu
from jax.experimental.pallas import tpu_sc as plsc

S, B, M = 256, 128, 21
V_CHAR, E_CHAR, E_WORD = 96, 30, 128
KW = 5
N = S * B
T_OUT = 19
EC_PAD = 32
EC_PK = EC_PAD // 2
MP = 24

NC, NS = 2, 16
NW = NC * NS
TOT = N * MP
CHUNK = 128
NCHUNK = TOT // (NW * CHUNK)
KG = 8
NG = NCHUNK // KG
GROWS = KG * CHUNK


def _sc_gather(table, idx):
    mesh = plsc.VectorSubcoreMesh(core_axis_name="c", subcore_axis_name="s")

    @functools.partial(
        pl.kernel,
        mesh=mesh,
        compiler_params=pltpu.CompilerParams(use_tc_tiling_on_sc=False),
        out_type=jax.ShapeDtypeStruct((TOT, EC_PK), jnp.int32),
        scratch_types=[
            pltpu.VMEM((NCHUNK, CHUNK), jnp.int32),
            pltpu.VMEM((GROWS, EC_PK), jnp.int32),
            pltpu.VMEM((GROWS, EC_PK), jnp.int32),
            pltpu.SemaphoreType.DMA,
            pltpu.SemaphoreType.DMA,
            pltpu.SemaphoreType.DMA,
            pltpu.SemaphoreType.DMA,
        ],
    )
    def run(table_hbm, idx_hbm, out_hbm, idx_v, buf_a, buf_b, sga, sgb, soa, sob):
        wid = lax.axis_index("s") * NC + lax.axis_index("c")
        pltpu.sync_copy(idx_hbm.at[wid], idx_v)
        base = wid * (NCHUNK * CHUNK)
        bufs, sgs, sos = (buf_a, buf_b), (sga, sgb), (soa, sob)

        def pair_body(i, carry):
            for p in range(2):
                g = 2 * i + p
                buf, sg, so = bufs[p], sgs[p], sos[p]

                @pl.when(g >= 2)
                def _():
                    pltpu.make_async_copy(
                        out_hbm.at[pl.ds(base, GROWS)], buf, so).wait()

                handles = [
                    pltpu.async_copy(
                        table_hbm.at[idx_v.at[g * KG + b]],
                        buf.at[pl.ds(b * CHUNK, CHUNK)], sg)
                    for b in range(KG)
                ]
                for h in handles:
                    h.wait()
                pltpu.async_copy(
                    buf, out_hbm.at[pl.ds(base + g * GROWS, GROWS)], so)
            return carry

        lax.fori_loop(0, NG // 2, pair_body, 0)
        for p in range(2):
            pltpu.make_async_copy(
                out_hbm.at[pl.ds(base, GROWS)], bufs[p], sos[p]).wait()

    return run(table, idx)


def _tc_body(x_ref, wc_ref, cb_ref, whw_ref, bhw_ref, o_ref, *, nb):
    R = nb * MP
    xi = x_ref[...]
    f_even = lax.bitcast_convert_type(xi << 16, jnp.float32)
    f_odd = lax.bitcast_convert_type(xi & jnp.int32(-65536), jnp.float32)
    x = jnp.concatenate([f_even, f_odd], axis=1).astype(jnp.bfloat16)
    zc = jnp.dot(x, wc_ref[...], preferred_element_type=jnp.float32)
    zpad = jnp.zeros((3, E_WORD), dtype=zc.dtype)
    y = cb_ref[...] + zc[:, 128:256]
    y = y + jnp.concatenate([zpad[:1], zc[:-1, 0:128]], axis=0)
    y = y + jnp.concatenate([zc[1:, 256:384], zpad[:1]], axis=0)
    y = y + jnp.concatenate([zc[2:, 384:512], zpad[:2]], axis=0)
    y = y + jnp.concatenate([zc[3:, 512:640], zpad[:3]], axis=0)
    xc = jnp.maximum(jnp.max(y.reshape(nb, MP, E_WORD)[:, :T_OUT], axis=1), 0.0)
    hw = jnp.dot(xc, whw_ref[...], preferred_element_type=jnp.float32) + bhw_ref[...]
    proj = jnp.maximum(hw[:, :E_WORD], 0.0)
    gate = 1.0 / (1.0 + jnp.exp(-hw[:, E_WORD:]))
    o_ref[...] = gate * proj + (1.0 - gate) * xc


def kernel(input, char_emb, conv_w, conv_b, w_proj, b_proj, w_gate, b_gate):
    nb = 128
    ce = jnp.pad(char_emb, ((0, 0), (0, EC_PAD - E_CHAR)))
    ce_u16 = lax.bitcast_convert_type(ce.astype(jnp.bfloat16), jnp.uint16)
    ce_pk = (ce_u16[:, 0::2].astype(jnp.uint32)
             | (ce_u16[:, 1::2].astype(jnp.uint32) << 16)).astype(jnp.int32)
    wcp = jnp.pad(conv_w, ((0, 0), (0, EC_PAD - E_CHAR), (0, 0)))
    wc = jnp.concatenate([wcp[w] for w in range(KW)], axis=1)
    wc = jnp.concatenate([wc[0::2], wc[1::2]], axis=0).astype(jnp.bfloat16)
    whw = jnp.concatenate([w_proj.T, w_gate.T], axis=1)
    bhw = jnp.concatenate([b_proj, b_gate])[None, :]
    cb = conv_b[None, :]

    idx_pad = jnp.pad(input.reshape(N, M), ((0, 0), (0, MP - M)))
    idx = idx_pad.reshape(NW, NCHUNK, CHUNK)
    x_pk = _sc_gather(ce_pk, idx)

    out = pl.pallas_call(
        functools.partial(_tc_body, nb=nb),
        grid=(N // nb,),
        in_specs=[
            pl.BlockSpec((nb * MP, EC_PK), lambda i: (i, 0)),
            pl.BlockSpec((EC_PAD, KW * E_WORD), lambda i: (0, 0)),
            pl.BlockSpec((1, E_WORD), lambda i: (0, 0)),
            pl.BlockSpec((E_WORD, 2 * E_WORD), lambda i: (0, 0)),
            pl.BlockSpec((1, 2 * E_WORD), lambda i: (0, 0)),
        ],
        out_specs=pl.BlockSpec((nb, E_WORD), lambda i: (i, 0)),
        out_shape=jax.ShapeDtypeStruct((N, E_WORD), jnp.float32),
    )(x_pk, wc, cb, whw, bhw)
    return out.reshape(S, B, E_WORD)


# speedup vs baseline: 4.1549x; 1.5923x over previous
import functools

import jax
import jax.numpy as jnp
from jax import lax
from jax.experimental import pallas as pl
from jax.experimental.pallas import tpu as pltpu
from jax.experimental.pallas import tpu_sc as plsc

S, B, M = 256, 128, 21
V_CHAR, E_CHAR, E_WORD = 96, 30, 128
KW = 5
N = S * B
T_OUT = 19
EC_PAD = 32
EC_PK = EC_PAD // 2
MP = 24

NC, NS = 2, 16
NW = NC * NS
TOT = N * MP
NW_CHARS = TOT // NW
GROWS = 1024
NG = NW_CHARS // GROWS


def _sc_gather(table, idx):
    mesh = plsc.VectorSubcoreMesh(core_axis_name="c", subcore_axis_name="s")

    @functools.partial(
        pl.kernel,
        mesh=mesh,
        compiler_params=pltpu.CompilerParams(
            use_tc_tiling_on_sc=False, needs_layout_passes=False),
        out_type=jax.ShapeDtypeStruct((TOT, EC_PK), jnp.int32),
        scratch_types=[
            pltpu.VMEM((V_CHAR * EC_PK,), jnp.int32),
            pltpu.VMEM((NW_CHARS,), jnp.int32),
            pltpu.VMEM((GROWS, EC_PK), jnp.int32),
            pltpu.VMEM((GROWS, EC_PK), jnp.int32),
            pltpu.SemaphoreType.DMA,
            pltpu.SemaphoreType.DMA,
        ],
    )
    def run(table_hbm, idx_hbm, out_hbm, table_v, idx_v, buf_a, buf_b, soa, sob):
        wid = lax.axis_index("s") * NC + lax.axis_index("c")
        pltpu.sync_copy(table_hbm, table_v)
        pltpu.sync_copy(idx_hbm.at[wid], idx_v)
        base = wid * NW_CHARS
        bufs, sos = (buf_a, buf_b), (soa, sob)
        lanes = lax.iota(jnp.int32, 16)

        def block16(buf, g, k):
            cv = idx_v[pl.ds(g * GROWS + k * 16, 16)]
            addr = cv * EC_PK
            rows = k * 16 + lanes
            for e in range(EC_PK):
                vals = plsc.load_gather(table_v, [addr + e])
                plsc.store_scatter(buf, [rows, lanes * 0 + e], vals)

        def pair_body(i, carry):
            for p in range(2):
                g = 2 * i + p
                buf, so = bufs[p], sos[p]

                @pl.when(g >= 2)
                def _():
                    pltpu.make_async_copy(
                        out_hbm.at[pl.ds(base, GROWS)], buf, so).wait()

                def kbody(k, c):
                    block16(buf, g, k)
                    return c

                lax.fori_loop(0, GROWS // 16, kbody, 0)
                pltpu.async_copy(
                    buf, out_hbm.at[pl.ds(base + g * GROWS, GROWS)], so)
            return carry

        lax.fori_loop(0, NG // 2, pair_body, 0)
        for p in range(2):
            pltpu.make_async_copy(
                out_hbm.at[pl.ds(base, GROWS)], bufs[p], sos[p]).wait()

    return run(table, idx)


def _tc_body(x_ref, wc_ref, cb_ref, whw_ref, bhw_ref, o_ref, *, nb):
    R = nb * MP
    xi = x_ref[...]
    f_even = lax.bitcast_convert_type(xi << 16, jnp.float32)
    f_odd = lax.bitcast_convert_type(xi & jnp.int32(-65536), jnp.float32)
    x = jnp.concatenate([f_even, f_odd], axis=1).astype(jnp.bfloat16)
    zc = jnp.dot(x, wc_ref[...], preferred_element_type=jnp.float32)
    zpad = jnp.zeros((3, E_WORD), dtype=zc.dtype)
    y = cb_ref[...] + zc[:, 128:256]
    y = y + jnp.concatenate([zpad[:1], zc[:-1, 0:128]], axis=0)
    y = y + jnp.concatenate([zc[1:, 256:384], zpad[:1]], axis=0)
    y = y + jnp.concatenate([zc[2:, 384:512], zpad[:2]], axis=0)
    y = y + jnp.concatenate([zc[3:, 512:640], zpad[:3]], axis=0)
    xc = jnp.maximum(jnp.max(y.reshape(nb, MP, E_WORD)[:, :T_OUT], axis=1), 0.0)
    hw = jnp.dot(xc, whw_ref[...], preferred_element_type=jnp.float32) + bhw_ref[...]
    proj = jnp.maximum(hw[:, :E_WORD], 0.0)
    gate = 1.0 / (1.0 + jnp.exp(-hw[:, E_WORD:]))
    o_ref[...] = gate * proj + (1.0 - gate) * xc


def kernel(input, char_emb, conv_w, conv_b, w_proj, b_proj, w_gate, b_gate):
    nb = 128
    ce = jnp.pad(char_emb, ((0, 0), (0, EC_PAD - E_CHAR)))
    ce_u16 = lax.bitcast_convert_type(ce.astype(jnp.bfloat16), jnp.uint16)
    ce_pk = (ce_u16[:, 0::2].astype(jnp.uint32)
             | (ce_u16[:, 1::2].astype(jnp.uint32) << 16)).astype(jnp.int32)
    wcp = jnp.pad(conv_w, ((0, 0), (0, EC_PAD - E_CHAR), (0, 0)))
    wc = jnp.concatenate([wcp[w] for w in range(KW)], axis=1)
    wc = jnp.concatenate([wc[0::2], wc[1::2]], axis=0).astype(jnp.bfloat16)
    whw = jnp.concatenate([w_proj.T, w_gate.T], axis=1)
    bhw = jnp.concatenate([b_proj, b_gate])[None, :]
    cb = conv_b[None, :]

    idx_pad = jnp.pad(input.reshape(N, M), ((0, 0), (0, MP - M)))
    idx = idx_pad.reshape(NW, NW_CHARS)
    x_pk = _sc_gather(ce_pk.reshape(-1), idx)

    out = pl.pallas_call(
        functools.partial(_tc_body, nb=nb),
        grid=(N // nb,),
        in_specs=[
            pl.BlockSpec((nb * MP, EC_PK), lambda i: (i, 0)),
            pl.BlockSpec((EC_PAD, KW * E_WORD), lambda i: (0, 0)),
            pl.BlockSpec((1, E_WORD), lambda i: (0, 0)),
            pl.BlockSpec((E_WORD, 2 * E_WORD), lambda i: (0, 0)),
            pl.BlockSpec((1, 2 * E_WORD), lambda i: (0, 0)),
        ],
        out_specs=pl.BlockSpec((nb, E_WORD), lambda i: (i, 0)),
        out_shape=jax.ShapeDtypeStruct((N, E_WORD), jnp.float32),
    )(x_pk, wc, cb, whw, bhw)
    return out.reshape(S, B, E_WORD)


# speedup vs baseline: 6.7895x; 1.6341x over previous
import functools

import jax
import jax.numpy as jnp
from jax import lax
from jax.experimental import pallas as pl
from jax.experimental.pallas import tpu as pltpu
from jax.experimental.pallas import tpu_sc as plsc

S, B, M = 256, 128, 21
V_CHAR, E_CHAR, E_WORD = 96, 30, 128
KW = 5
N = S * B
T_OUT = 19
EC_PAD = 32
EC_PK = EC_PAD // 2
MP = 24

NC, NS = 2, 16
NW = NC * NS
TOT = N * MP
NW_CHARS = TOT // NW
GROWS = 1024
NG = NW_CHARS // GROWS


def _sc_gather(table, idx):
    mesh = plsc.VectorSubcoreMesh(core_axis_name="c", subcore_axis_name="s")

    @functools.partial(
        pl.kernel,
        mesh=mesh,
        compiler_params=pltpu.CompilerParams(
            use_tc_tiling_on_sc=False, needs_layout_passes=False),
        out_type=jax.ShapeDtypeStruct((TOT, EC_PK), jnp.int32),
        scratch_types=[
            pltpu.VMEM((V_CHAR * EC_PK,), jnp.int32),
            pltpu.VMEM((NW_CHARS,), jnp.int32),
            pltpu.VMEM((GROWS, EC_PK), jnp.int32),
            pltpu.VMEM((GROWS, EC_PK), jnp.int32),
            pltpu.SemaphoreType.DMA,
            pltpu.SemaphoreType.DMA,
        ],
    )
    def run(table_hbm, idx_hbm, out_hbm, table_v, idx_v, buf_a, buf_b, soa, sob):
        wid = lax.axis_index("s") * NC + lax.axis_index("c")
        pltpu.sync_copy(table_hbm, table_v)
        pltpu.sync_copy(idx_hbm.at[wid], idx_v)
        base = wid * NW_CHARS
        bufs, sos = (buf_a, buf_b), (soa, sob)
        lanes = lax.iota(jnp.int32, 16)

        def block16(buf, g, k):
            cv = idx_v[pl.ds(g * GROWS + k * 16, 16)]
            addr = cv * EC_PK
            rows = k * 16 + lanes
            for e in range(EC_PK):
                vals = plsc.load_gather(table_v, [addr + e])
                plsc.store_scatter(buf, [rows, lanes * 0 + e], vals)

        def pair_body(i, carry):
            for p in range(2):
                g = 2 * i + p
                buf, so = bufs[p], sos[p]

                @pl.when(g >= 2)
                def _():
                    pltpu.make_async_copy(
                        out_hbm.at[pl.ds(base, GROWS)], buf, so).wait()

                def kbody(k, c):
                    block16(buf, g, k)
                    return c

                lax.fori_loop(0, GROWS // 16, kbody, 0)
                pltpu.async_copy(
                    buf, out_hbm.at[pl.ds(base + g * GROWS, GROWS)], so)
            return carry

        lax.fori_loop(0, NG // 2, pair_body, 0)
        for p in range(2):
            pltpu.make_async_copy(
                out_hbm.at[pl.ds(base, GROWS)], bufs[p], sos[p]).wait()

    return run(table, idx)


def _tc_body(x_ref, wc_ref, cb_ref, whw_ref, bhw_ref, o_ref, *, nb):
    xi = x_ref[...]
    f_even = lax.bitcast_convert_type(xi << 16, jnp.float32)
    f_odd = lax.bitcast_convert_type(xi & jnp.int32(-65536), jnp.float32)
    x = jnp.concatenate([f_even, f_odd], axis=1).astype(jnp.bfloat16)
    half = MP * EC_PK
    m = None
    for t in range(T_OUT):
        xt = jnp.concatenate(
            [x[:, 16 * t:16 * t + 80],
             x[:, half + 16 * t:half + 16 * t + 80]], axis=1)
        zt = jnp.dot(xt, wc_ref[...], preferred_element_type=jnp.float32)
        m = zt if m is None else jnp.maximum(m, zt)
    xc = jnp.maximum(m + cb_ref[...], 0.0)
    hw = jnp.dot(xc, whw_ref[...], preferred_element_type=jnp.float32) + bhw_ref[...]
    proj = jnp.maximum(hw[:, :E_WORD], 0.0)
    gate = 1.0 / (1.0 + jnp.exp(-hw[:, E_WORD:]))
    o_ref[...] = gate * proj + (1.0 - gate) * xc


def kernel(input, char_emb, conv_w, conv_b, w_proj, b_proj, w_gate, b_gate):
    nb = 128
    ce = jnp.pad(char_emb, ((0, 0), (0, EC_PAD - E_CHAR)))
    ce_u16 = lax.bitcast_convert_type(ce.astype(jnp.bfloat16), jnp.uint16)
    ce_pk = (ce_u16[:, 0::2].astype(jnp.uint32)
             | (ce_u16[:, 1::2].astype(jnp.uint32) << 16)).astype(jnp.int32)
    wcp = jnp.pad(conv_w, ((0, 0), (0, EC_PAD - E_CHAR), (0, 0)))
    wc = jnp.concatenate([wcp[:, 0::2, :].reshape(KW * EC_PK, E_WORD),
                          wcp[:, 1::2, :].reshape(KW * EC_PK, E_WORD)],
                         axis=0).astype(jnp.bfloat16)
    whw = jnp.concatenate([w_proj.T, w_gate.T], axis=1)
    bhw = jnp.concatenate([b_proj, b_gate])[None, :]
    cb = conv_b[None, :]

    idx_pad = jnp.pad(input.reshape(N, M), ((0, 0), (1, MP - M - 1)))
    idx = idx_pad.reshape(NW, NW_CHARS)
    x_pk = _sc_gather(ce_pk.reshape(-1), idx)
    x_pk = x_pk.reshape(N, MP * EC_PK)

    out = pl.pallas_call(
        functools.partial(_tc_body, nb=nb),
        grid=(N // nb,),
        in_specs=[
            pl.BlockSpec((nb, MP * EC_PK), lambda i: (i, 0)),
            pl.BlockSpec((KW * EC_PK * 2, E_WORD), lambda i: (0, 0)),
            pl.BlockSpec((1, E_WORD), lambda i: (0, 0)),
            pl.BlockSpec((E_WORD, 2 * E_WORD), lambda i: (0, 0)),
            pl.BlockSpec((1, 2 * E_WORD), lambda i: (0, 0)),
        ],
        out_specs=pl.BlockSpec((nb, E_WORD), lambda i: (i, 0)),
        out_shape=jax.ShapeDtypeStruct((N, E_WORD), jnp.float32),
    )(x_pk, wc, cb, whw, bhw)
    return out.reshape(S, B, E_WORD)


# speedup vs baseline: 6.8926x; 1.0152x over previous
import functools

import jax
import jax.numpy as jnp
from jax import lax
from jax.experimental import pallas as pl
from jax.experimental.pallas import tpu as pltpu
from jax.experimental.pallas import tpu_sc as plsc

S, B, M = 256, 128, 21
V_CHAR, E_CHAR, E_WORD = 96, 30, 128
KW = 5
N = S * B
T_OUT = 19
EC_PAD = 32
EC_PK = EC_PAD // 2
MP = 24

NC, NS = 2, 16
NW = NC * NS
GROWS = 1024
NCHUNK = 2
NWRD = N // NCHUNK
NW_CHARS = NWRD * MP // NW
NG = NW_CHARS // GROWS


def _sc_gather(table, idx):
    mesh = plsc.VectorSubcoreMesh(core_axis_name="c", subcore_axis_name="s")

    @functools.partial(
        pl.kernel,
        mesh=mesh,
        compiler_params=pltpu.CompilerParams(
            use_tc_tiling_on_sc=False, needs_layout_passes=False),
        out_type=jax.ShapeDtypeStruct((NWRD * MP, EC_PK), jnp.int32),
        scratch_types=[
            pltpu.VMEM((V_CHAR * EC_PK,), jnp.int32),
            pltpu.VMEM((NW_CHARS,), jnp.int32),
            pltpu.VMEM((GROWS, EC_PK), jnp.int32),
            pltpu.VMEM((GROWS, EC_PK), jnp.int32),
            pltpu.SemaphoreType.DMA,
            pltpu.SemaphoreType.DMA,
        ],
    )
    def run(table_hbm, idx_hbm, out_hbm, table_v, idx_v, buf_a, buf_b, soa, sob):
        wid = lax.axis_index("s") * NC + lax.axis_index("c")
        pltpu.sync_copy(table_hbm, table_v)
        pltpu.sync_copy(idx_hbm.at[wid], idx_v)
        base = wid * NW_CHARS
        bufs, sos = (buf_a, buf_b), (soa, sob)
        lanes = lax.iota(jnp.int32, 16)

        def block16(buf, g, k):
            cv = idx_v[pl.ds(g * GROWS + k * 16, 16)]
            addr = cv * EC_PK
            rows = k * 16 + lanes
            for e in range(EC_PK):
                vals = plsc.load_gather(table_v, [addr + e])
                plsc.store_scatter(buf, [rows, lanes * 0 + e], vals)

        def pair_body(i, carry):
            for p in range(2):
                g = 2 * i + p
                buf, so = bufs[p], sos[p]

                @pl.when(g >= 2)
                def _():
                    pltpu.make_async_copy(
                        out_hbm.at[pl.ds(base, GROWS)], buf, so).wait()

                def kbody(k, c):
                    block16(buf, g, k)
                    return c

                lax.fori_loop(0, GROWS // 16, kbody, 0)
                pltpu.async_copy(
                    buf, out_hbm.at[pl.ds(base + g * GROWS, GROWS)], so)
            return carry

        lax.fori_loop(0, NG // 2, pair_body, 0)
        for p in range(2):
            pltpu.make_async_copy(
                out_hbm.at[pl.ds(base, GROWS)], bufs[p], sos[p]).wait()

    return run(table, idx)


def _tc_body(x_ref, wc_ref, cb_ref, whw_ref, bhw_ref, o_ref, *, nb):
    xi = x_ref[...]
    xe = lax.bitcast_convert_type(xi << 16, jnp.float32).astype(jnp.bfloat16)
    xo = lax.bitcast_convert_type(
        xi & jnp.int32(-65536), jnp.float32).astype(jnp.bfloat16)
    m = None
    for t in range(T_OUT):
        xt = jnp.concatenate(
            [xe[:, 16 * t:16 * t + 80], xo[:, 16 * t:16 * t + 80]],
            axis=1)
        zt = jnp.dot(xt, wc_ref[...], preferred_element_type=jnp.float32)
        m = zt if m is None else jnp.maximum(m, zt)
    xc = jnp.maximum(m + cb_ref[...], 0.0)
    hw = jnp.dot(xc, whw_ref[...], preferred_element_type=jnp.float32) + bhw_ref[...]
    proj = jnp.maximum(hw[:, :E_WORD], 0.0)
    gate = 1.0 / (1.0 + jnp.exp(-hw[:, E_WORD:]))
    o_ref[...] = gate * proj + (1.0 - gate) * xc


def kernel(input, char_emb, conv_w, conv_b, w_proj, b_proj, w_gate, b_gate):
    nb = 128
    ce = jnp.pad(char_emb, ((0, 0), (0, EC_PAD - E_CHAR)))
    ce_u16 = lax.bitcast_convert_type(ce.astype(jnp.bfloat16), jnp.uint16)
    ce_pk = (ce_u16[:, 0::2].astype(jnp.uint32)
             | (ce_u16[:, 1::2].astype(jnp.uint32) << 16)).astype(jnp.int32)
    wcp = jnp.pad(conv_w, ((0, 0), (0, EC_PAD - E_CHAR), (0, 0)))
    wc = jnp.concatenate([wcp[:, 0::2, :].reshape(KW * EC_PK, E_WORD),
                          wcp[:, 1::2, :].reshape(KW * EC_PK, E_WORD)],
                         axis=0).astype(jnp.bfloat16)
    whw = jnp.concatenate([w_proj.T, w_gate.T], axis=1)
    bhw = jnp.concatenate([b_proj, b_gate])[None, :]
    cb = conv_b[None, :]

    idx_pad = jnp.pad(input.reshape(N, M), ((0, 0), (1, MP - M - 1)))

    tc_call = pl.pallas_call(
        functools.partial(_tc_body, nb=nb),
        grid=(NWRD // nb,),
        in_specs=[
            pl.BlockSpec((nb, MP * EC_PK), lambda i: (i, 0)),
            pl.BlockSpec((KW * EC_PK * 2, E_WORD), lambda i: (0, 0)),
            pl.BlockSpec((1, E_WORD), lambda i: (0, 0)),
            pl.BlockSpec((E_WORD, 2 * E_WORD), lambda i: (0, 0)),
            pl.BlockSpec((1, 2 * E_WORD), lambda i: (0, 0)),
        ],
        out_specs=pl.BlockSpec((nb, E_WORD), lambda i: (i, 0)),
        out_shape=jax.ShapeDtypeStruct((NWRD, E_WORD), jnp.float32),
    )
    outs = []
    for c in range(NCHUNK):
        idx_c = lax.slice_in_dim(idx_pad, c * NWRD, (c + 1) * NWRD)
        x_pk = _sc_gather(ce_pk.reshape(-1), idx_c.reshape(NW, NW_CHARS))
        outs.append(tc_call(x_pk.reshape(NWRD, MP * EC_PK), wc, cb, whw, bhw))
    return jnp.concatenate(outs, axis=0).reshape(S, B, E_WORD)


# speedup vs baseline: 6.8929x; 1.0000x over previous
import functools

import jax
import jax.numpy as jnp
from jax import lax
from jax.experimental import pallas as pl
from jax.experimental.pallas import tpu as pltpu
from jax.experimental.pallas import tpu_sc as plsc

S, B, M = 256, 128, 21
V_CHAR, E_CHAR, E_WORD = 96, 30, 128
KW = 5
N = S * B
T_OUT = 19
EC_PAD = 32
EC_PK = EC_PAD // 2
MP = 24

NC, NS = 2, 16
NW = NC * NS
GROWS = 1024
NCHUNK = 2
NWRD = N // NCHUNK
NW_CHARS = NWRD * MP // NW
NG = NW_CHARS // GROWS


def _sc_gather(table, idx):
    mesh = plsc.VectorSubcoreMesh(core_axis_name="c", subcore_axis_name="s")

    @functools.partial(
        pl.kernel,
        mesh=mesh,
        compiler_params=pltpu.CompilerParams(
            use_tc_tiling_on_sc=False, needs_layout_passes=False),
        out_type=jax.ShapeDtypeStruct((NWRD * MP, EC_PK), jnp.int32),
        scratch_types=[
            pltpu.VMEM((V_CHAR * EC_PK,), jnp.int32),
            pltpu.VMEM((NW_CHARS,), jnp.int32),
            pltpu.VMEM((GROWS, EC_PK), jnp.int32),
            pltpu.VMEM((GROWS, EC_PK), jnp.int32),
            pltpu.SemaphoreType.DMA,
            pltpu.SemaphoreType.DMA,
        ],
    )
    def run(table_hbm, idx_hbm, out_hbm, table_v, idx_v, buf_a, buf_b, soa, sob):
        wid = lax.axis_index("s") * NC + lax.axis_index("c")
        pltpu.sync_copy(table_hbm, table_v)
        pltpu.sync_copy(idx_hbm.at[wid], idx_v)
        base = wid * NW_CHARS
        bufs, sos = (buf_a, buf_b), (soa, sob)
        lanes = lax.iota(jnp.int32, 16)

        def block16(buf, g, k):
            cv = idx_v[pl.ds(g * GROWS + k * 16, 16)]
            addr = cv * EC_PK
            rows = k * 16 + lanes
            for e in range(EC_PK):
                vals = plsc.load_gather(table_v, [addr + e])
                plsc.store_scatter(buf, [rows, lanes * 0 + e], vals)

        def pair_body(i, carry):
            for p in range(2):
                g = 2 * i + p
                buf, so = bufs[p], sos[p]

                @pl.when(g >= 2)
                def _():
                    pltpu.make_async_copy(
                        out_hbm.at[pl.ds(base, GROWS)], buf, so).wait()

                def kbody(k, c):
                    block16(buf, g, k)
                    return c

                lax.fori_loop(0, GROWS // 16, kbody, 0)
                pltpu.async_copy(
                    buf, out_hbm.at[pl.ds(base + g * GROWS, GROWS)], so)
            return carry

        lax.fori_loop(0, NG // 2, pair_body, 0)
        for p in range(2):
            pltpu.make_async_copy(
                out_hbm.at[pl.ds(base, GROWS)], bufs[p], sos[p]).wait()

    return run(table, idx)


def _tc_body(x_ref, wc_ref, cb_ref, whw_ref, bhw_ref, o_ref, *, nb):
    xi = x_ref[...]
    xe = lax.bitcast_convert_type(xi << 16, jnp.float32).astype(jnp.bfloat16)
    xo = lax.bitcast_convert_type(
        xi & jnp.int32(-65536), jnp.float32).astype(jnp.bfloat16)
    m = None
    for t in range(T_OUT):
        xt = jnp.concatenate(
            [xe[:, 16 * t:16 * t + 80], xo[:, 16 * t:16 * t + 80]],
            axis=1)
        zt = jnp.dot(xt, wc_ref[...], preferred_element_type=jnp.float32)
        m = zt if m is None else jnp.maximum(m, zt)
    xc = jnp.maximum(m + cb_ref[...], 0.0)
    hw = jnp.dot(xc, whw_ref[...], preferred_element_type=jnp.float32) + bhw_ref[...]
    proj = jnp.maximum(hw[:, :E_WORD], 0.0)
    gate = 1.0 / (1.0 + jnp.exp(-hw[:, E_WORD:]))
    o_ref[...] = gate * proj + (1.0 - gate) * xc


def kernel(input, char_emb, conv_w, conv_b, w_proj, b_proj, w_gate, b_gate):
    nb = 128
    ce = jnp.pad(char_emb, ((0, 0), (0, EC_PAD - E_CHAR)))
    ce_u16 = lax.bitcast_convert_type(ce.astype(jnp.bfloat16), jnp.uint16)
    ce_pk = (ce_u16[:, 0::2].astype(jnp.uint32)
             | (ce_u16[:, 1::2].astype(jnp.uint32) << 16)).astype(jnp.int32)
    wcp = jnp.pad(conv_w, ((0, 0), (0, EC_PAD - E_CHAR), (0, 0)))
    wc = jnp.concatenate([wcp[:, 0::2, :].reshape(KW * EC_PK, E_WORD),
                          wcp[:, 1::2, :].reshape(KW * EC_PK, E_WORD)],
                         axis=0).astype(jnp.bfloat16)
    whw = jnp.concatenate([w_proj.T, w_gate.T], axis=1)
    bhw = jnp.concatenate([b_proj, b_gate])[None, :]
    cb = conv_b[None, :]

    idx_pad = jnp.pad(input.reshape(N, M), ((0, 0), (1, MP - M - 1)))

    tc_call = pl.pallas_call(
        functools.partial(_tc_body, nb=nb),
        grid=(NWRD // nb,),
        in_specs=[
            pl.BlockSpec((nb, MP * EC_PK), lambda i: (i, 0)),
            pl.BlockSpec((KW * EC_PK * 2, E_WORD), lambda i: (0, 0)),
            pl.BlockSpec((1, E_WORD), lambda i: (0, 0)),
            pl.BlockSpec((E_WORD, 2 * E_WORD), lambda i: (0, 0)),
            pl.BlockSpec((1, 2 * E_WORD), lambda i: (0, 0)),
        ],
        out_specs=pl.BlockSpec((nb, E_WORD), lambda i: (i, 0)),
        out_shape=jax.ShapeDtypeStruct((NWRD, E_WORD), jnp.float32),
    )
    gathered = []
    for c in range(NCHUNK):
        idx_c = lax.slice_in_dim(idx_pad, c * NWRD, (c + 1) * NWRD)
        x_pk = _sc_gather(ce_pk.reshape(-1), idx_c.reshape(NW, NW_CHARS))
        gathered.append(x_pk.reshape(NWRD, MP * EC_PK))
    outs = [tc_call(x_pk, wc, cb, whw, bhw) for x_pk in gathered]
    return jnp.concatenate(outs, axis=0).reshape(S, B, E_WORD)
